# baseline (speedup 1.0000x reference)
import functools

import jax
import jax.numpy as jnp
from jax import lax
from jax.experimental import pallas as pl
from jax.experimental.pallas import tpu as pltpu
from jax.experimental.pallas import tpu_sc as plsc

_B = 16384
_D = 64
_NC = 2
_NS = 16
_L = 16
_NW = _NC * _NS
_NCLS = 1000000
_SW = 256
_NSTRIP = (_NCLS + _SW - 1) // _SW
_SPW = (_NSTRIP + _NW - 1) // _NW
_LAST = _NSTRIP - 1
_CAP = 4096
_CHUNK = 2048
_NCHUNK = _B // _CHUNK
_NBUF = 3


@functools.partial(
    pl.kernel,
    out_type=jax.ShapeDtypeStruct((_NW, 128), jnp.float32),
    mesh=plsc.VectorSubcoreMesh(core_axis_name="c", subcore_axis_name="s"),
    compiler_params=pltpu.CompilerParams(
        use_tc_tiling_on_sc=True,
        needs_layout_passes=False,
        disable_bounds_checks=True,
    ),
    scratch_types=[
        pltpu.VMEM((_CHUNK,), jnp.int32),
        pltpu.VMEM((_CAP,), jnp.int32),
        pltpu.VMEM((_CAP,), jnp.int32),
        pltpu.SMEM((_SPW + 8,), jnp.int32),
        [pltpu.VMEM((_D, _SW), jnp.float32) for _ in range(_NBUF)],
        pltpu.VMEM((8, 128), jnp.float32),
        pltpu.VMEM((128,), jnp.float32),
        pltpu.VMEM_SHARED((_B // 16, 8, 128), jnp.float32),
        pltpu.SemaphoreType.DMA,
        pltpu.SemaphoreType.DMA,
    ],
)
def _center_loss_partials(label_hbm, feat_hbm, centers_t_hbm, out_hbm,
                          lab_v, mpk_v, bpk_v, cnt_s, bufs,
                          fbuf_v, acc_v, feat_sh, sem, sem2):
    cid = lax.axis_index("c")
    sid = lax.axis_index("s")
    wid = sid * _NC + cid
    lanes = lax.iota(jnp.int32, _L)
    lane0 = lanes == 0

    @pl.when(sid == 0)
    def _():
        pltpu.sync_copy(feat_hbm, feat_sh)

    plsc.subcore_barrier()

    lo_strip = wid * _SPW
    lo = lo_strip * _SW
    hi = lo + _SPW * _SW
    zero = jnp.zeros((_L,), jnp.float32)

    def strip_src(s):
        s_eff = jnp.minimum(lo_strip + s, _LAST)
        off = pl.multiple_of(s_eff * _SW, _SW)
        return centers_t_hbm.at[:, pl.ds(off, _SW)]

    def wait_strip(buf):
        pltpu.make_async_copy(strip_src(jnp.int32(0)), buf, sem).wait()

    def segment(carry):
        h0, accs = carry

        def scan_chunk_cond(c):
            h, pos = c
            return (h < _NCHUNK) & (pos <= _CAP - _CHUNK)

        def scan_chunk(c):
            h, pos = c
            pltpu.sync_copy(label_hbm.at[pl.ds(h * _CHUNK, _CHUNK)], lab_v)

            def scan_body(k, pos):
                lv = lab_v[pl.ds(k * _L, _L)]
                m = (lv >= lo) & (lv < hi)
                pk = ((lv - lo) << 14) | (lanes + (h * _CHUNK + k * _L))
                plsc.store_compressed(mpk_v.at[pl.ds(pos, _L)], pk, mask=m)
                return pos + plsc.all_reduce_population_count(m)[0]

            pos = lax.fori_loop(0, _CHUNK // _L, scan_body, pos)
            return h + 1, pos

        h1, nm = lax.while_loop(scan_chunk_cond, scan_chunk,
                                (h0, jnp.int32(0)))

        def zero_body(s, carry):
            cnt_s[s] = 0
            return carry

        lax.fori_loop(0, _SPW + 1, zero_body, 0)

        def hist_body(m, carry):
            s = mpk_v[pl.ds(m, _L)][0] >> 22
            cnt_s[s] = cnt_s[s] + 1
            return carry

        lax.fori_loop(0, nm, hist_body, 0)

        def pfx_body(s, run):
            t = cnt_s[s]
            cnt_s[s] = run
            return run + t

        lax.fori_loop(0, _SPW + 1, pfx_body, jnp.int32(0))

        def ins_body(m, carry):
            pk = mpk_v[pl.ds(m, _L)][0]
            s = pk >> 22
            p = cnt_s[s]
            cnt_s[s] = p + 1
            plsc.store_scatter(bpk_v, [jnp.broadcast_to(p, (_L,))],
                               jnp.broadcast_to(pk, (_L,)), mask=lane0)
            return carry

        lax.fori_loop(0, nm, ins_body, 0)

        def process(strip_v, s, pe, accs):
            ce = cnt_s[s]

            def mbody(m, accs):
                pk = bpk_v[pl.ds(m, _L)][0]
                r = pk & (_B - 1)
                cm = jnp.broadcast_to((pk >> 14) & (_SW - 1), (_L,))
                pltpu.sync_copy(feat_sh.at[r // 16], fbuf_v)
                frow = (r // 2) % 8
                foff = (r % 2) * _D
                new = []
                for q in range(_D // _L):
                    c = plsc.load_gather(strip_v, [lanes + q * _L, cm])
                    f = fbuf_v[frow, pl.ds(foff + q * _L, _L)]
                    d = f - c
                    new.append(accs[q] + d * d)
                return tuple(new)

            accs = lax.fori_loop(pe, ce, mbody, accs)
            return ce, accs

        for j in range(_NBUF):
            pltpu.async_copy(strip_src(jnp.int32(j)), bufs[j], sem)

        def pipe_body(i, carry):
            pe, accs = carry
            s0 = i * _NBUF
            for j in range(_NBUF):
                wait_strip(bufs[j])
                pe, accs = process(bufs[j], s0 + j, pe, accs)
                pltpu.async_copy(strip_src(s0 + j + _NBUF), bufs[j], sem)
            return pe, accs

        ntr = _SPW // _NBUF
        pe, accs = lax.fori_loop(0, ntr, pipe_body, (jnp.int32(0), accs))
        for j in range(_NBUF):
            wait_strip(bufs[j])
            if ntr * _NBUF + j < _SPW:
                pe, accs = process(bufs[j], jnp.int32(ntr * _NBUF + j),
                                   pe, accs)

        return h1, accs

    def seg_cond(carry):
        h, _ = carry
        return h < _NCHUNK

    _, accs = lax.while_loop(seg_cond, segment,
                             (jnp.int32(0), (zero, zero, zero, zero)))

    zero16 = jnp.zeros((_L,), jnp.float32)
    for z in range(8):
        acc_v[pl.ds(z * _L, _L)] = zero16
    acc_v[pl.ds(0, _L)] = (accs[0] + accs[1]) + (accs[2] + accs[3])
    pltpu.sync_copy(acc_v, out_hbm.at[wid])


def kernel(label, feat, centers):
    feat2 = feat.reshape(_B // 16, 8, 128)
    partials = _center_loss_partials(label, feat2, centers.T)
    return jnp.sum(partials) * (0.5 / _B)


# speedup vs baseline: 1.0212x; 1.0212x over previous
import functools

import jax
import jax.numpy as jnp
from jax import lax
from jax.experimental import pallas as pl
from jax.experimental.pallas import tpu as pltpu
from jax.experimental.pallas import tpu_sc as plsc

_B = 16384
_D = 64
_NC = 2
_NS = 16
_L = 16
_NW = _NC * _NS
_NCLS = 1000000
_SW = 128
_NSTRIP = (_NCLS + _SW - 1) // _SW
_SPW = (_NSTRIP + _NW - 1) // _NW
_LAST = _NSTRIP - 1
_CAP = 4096
_CHUNK = 2048
_NCHUNK = _B // _CHUNK
_NBUF = 6


@functools.partial(
    pl.kernel,
    out_type=jax.ShapeDtypeStruct((_NW, 128), jnp.float32),
    mesh=plsc.VectorSubcoreMesh(core_axis_name="c", subcore_axis_name="s"),
    compiler_params=pltpu.CompilerParams(
        use_tc_tiling_on_sc=True,
        needs_layout_passes=False,
        disable_bounds_checks=True,
    ),
    scratch_types=[
        pltpu.VMEM((_CHUNK,), jnp.int32),
        pltpu.VMEM((_CAP,), jnp.int32),
        pltpu.VMEM((_CAP,), jnp.int32),
        pltpu.SMEM((_SPW + 8,), jnp.int32),
        pltpu.SMEM((_SPW + 8,), jnp.int32),
        [pltpu.VMEM((_D, _SW), jnp.float32) for _ in range(_NBUF)],
        pltpu.VMEM((8, 128), jnp.float32),
        pltpu.VMEM((128,), jnp.float32),
        pltpu.VMEM_SHARED((_B // 16, 8, 128), jnp.float32),
        pltpu.SemaphoreType.DMA,
        pltpu.SemaphoreType.DMA,
    ],
)
def _center_loss_partials(label_hbm, feat_hbm, centers_t_hbm, out_hbm,
                          lab_v, mpk_v, bpk_v, cnt_s, slist_s, bufs,
                          fbuf_v, acc_v, feat_sh, sem, sem2):
    cid = lax.axis_index("c")
    sid = lax.axis_index("s")
    wid = sid * _NC + cid
    lanes = lax.iota(jnp.int32, _L)
    lane0 = lanes == 0

    @pl.when(sid == 0)
    def _():
        pltpu.sync_copy(feat_hbm, feat_sh)

    plsc.subcore_barrier()

    lo_strip = wid * _SPW
    lo = lo_strip * _SW
    hi = lo + _SPW * _SW
    zero = jnp.zeros((_L,), jnp.float32)

    def strip_src(s):
        s_eff = jnp.minimum(lo_strip + s, _LAST)
        off = pl.multiple_of(s_eff * _SW, _SW)
        return centers_t_hbm.at[:, pl.ds(off, _SW)]

    def wait_strip(buf):
        pltpu.make_async_copy(strip_src(jnp.int32(0)), buf, sem).wait()

    def segment(carry):
        h0, accs = carry

        def scan_chunk_cond(c):
            h, pos = c
            return (h < _NCHUNK) & (pos <= _CAP - _CHUNK)

        def scan_chunk(c):
            h, pos = c
            pltpu.sync_copy(label_hbm.at[pl.ds(h * _CHUNK, _CHUNK)], lab_v)

            def scan_body(k, pos):
                lv = lab_v[pl.ds(k * _L, _L)]
                m = (lv >= lo) & (lv < hi)
                pk = ((lv - lo) << 14) | (lanes + (h * _CHUNK + k * _L))
                plsc.store_compressed(mpk_v.at[pl.ds(pos, _L)], pk, mask=m)
                return pos + plsc.all_reduce_population_count(m)[0]

            pos = lax.fori_loop(0, _CHUNK // _L, scan_body, pos)
            return h + 1, pos

        h1, nm = lax.while_loop(scan_chunk_cond, scan_chunk,
                                (h0, jnp.int32(0)))

        def zero_body(s, carry):
            cnt_s[s] = 0
            return carry

        lax.fori_loop(0, _SPW + 1, zero_body, 0)

        def hist_body(m, carry):
            s = mpk_v[pl.ds(m, _L)][0] >> 21
            cnt_s[s] = cnt_s[s] + 1
            return carry

        lax.fori_loop(0, nm, hist_body, 0)

        def pfx_body(s, run):
            t = cnt_s[s]
            cnt_s[s] = run
            return run + t

        lax.fori_loop(0, _SPW + 1, pfx_body, jnp.int32(0))

        def ins_body(m, carry):
            pk = mpk_v[pl.ds(m, _L)][0]
            s = pk >> 21
            p = cnt_s[s]
            cnt_s[s] = p + 1
            plsc.store_scatter(bpk_v, [jnp.broadcast_to(p, (_L,))],
                               jnp.broadcast_to(pk, (_L,)), mask=lane0)
            return carry

        lax.fori_loop(0, nm, ins_body, 0)

        def build_body(s, carry):
            nu, prev = carry
            ce = cnt_s[s]

            @pl.when(ce > prev)
            def _():
                slist_s[nu] = s

            return jnp.where(ce > prev, nu + 1, nu), ce

        nu, _ = lax.fori_loop(0, _SPW, build_body,
                              (jnp.int32(0), jnp.int32(0)))

        @pl.when(nu == 0)
        def _():
            slist_s[0] = 0

        nu = jnp.maximum(nu, 1)

        def process(strip_v, ce, pe, accs):

            def mbody(m, accs):
                pk = bpk_v[pl.ds(m, _L)][0]
                r = pk & (_B - 1)
                cm = jnp.broadcast_to((pk >> 14) & (_SW - 1), (_L,))
                pltpu.sync_copy(feat_sh.at[r // 16], fbuf_v)
                frow = (r // 2) % 8
                foff = (r % 2) * _D
                new = []
                for q in range(_D // _L):
                    c = plsc.load_gather(strip_v, [lanes + q * _L, cm])
                    f = fbuf_v[frow, pl.ds(foff + q * _L, _L)]
                    d = f - c
                    new.append(accs[q] + d * d)
                return tuple(new)

            accs = lax.fori_loop(pe, ce, mbody, accs)
            return ce, accs

        def sget(u):
            return slist_s[jnp.minimum(u, nu - 1)]

        for j in range(_NBUF):
            pltpu.async_copy(strip_src(sget(jnp.int32(j))), bufs[j], sem)

        def pipe_body(i, carry):
            pe, accs = carry
            u0 = i * _NBUF
            for j in range(_NBUF):
                wait_strip(bufs[j])
                pe, accs = process(bufs[j], cnt_s[sget(u0 + j)], pe, accs)
                pltpu.async_copy(strip_src(sget(u0 + j + _NBUF)), bufs[j],
                                 sem)
            return pe, accs

        ntr = (nu + _NBUF - 1) // _NBUF
        pe, accs = lax.fori_loop(0, ntr, pipe_body, (jnp.int32(0), accs))
        for j in range(_NBUF):
            wait_strip(bufs[j])

        return h1, accs

    def seg_cond(carry):
        h, _ = carry
        return h < _NCHUNK

    _, accs = lax.while_loop(seg_cond, segment,
                             (jnp.int32(0), (zero, zero, zero, zero)))

    zero16 = jnp.zeros((_L,), jnp.float32)
    for z in range(8):
        acc_v[pl.ds(z * _L, _L)] = zero16
    acc_v[pl.ds(0, _L)] = (accs[0] + accs[1]) + (accs[2] + accs[3])
    pltpu.sync_copy(acc_v, out_hbm.at[wid])


def kernel(label, feat, centers):
    feat2 = feat.reshape(_B // 16, 8, 128)
    partials = _center_loss_partials(label, feat2, centers.T)
    return jnp.sum(partials) * (0.5 / _B)


# speedup vs baseline: 1.1592x; 1.1352x over previous
import functools

import jax
import jax.numpy as jnp
from jax import lax
from jax.experimental import pallas as pl
from jax.experimental.pallas import tpu as pltpu
from jax.experimental.pallas import tpu_sc as plsc

_B = 16384
_D = 64
_NC = 2
_NS = 16
_L = 16
_NW = _NC * _NS
_NCLS = 1000000
_SW = 128
_NSTRIP = (_NCLS + _SW - 1) // _SW
_SPW = (_NSTRIP + _NW - 1) // _NW
_LAST = _NSTRIP - 1
_CAP = 4096
_CHUNK = 2048
_NCHUNK = _B // _CHUNK
_NBUF = 6


@functools.partial(
    pl.kernel,
    out_type=jax.ShapeDtypeStruct((_NW, 128), jnp.float32),
    mesh=plsc.VectorSubcoreMesh(core_axis_name="c", subcore_axis_name="s"),
    compiler_params=pltpu.CompilerParams(
        use_tc_tiling_on_sc=True,
        needs_layout_passes=False,
        disable_bounds_checks=True,
    ),
    scratch_types=[
        pltpu.VMEM((_CHUNK,), jnp.int32),
        pltpu.VMEM((_CAP,), jnp.int32),
        pltpu.VMEM((_CAP,), jnp.int32),
        pltpu.SMEM((_SPW + 8,), jnp.int32),
        pltpu.SMEM((_SPW + 8,), jnp.int32),
        [pltpu.VMEM((_D, _SW), jnp.float32) for _ in range(_NBUF)],
        pltpu.VMEM((128,), jnp.float32),
        pltpu.VMEM((128,), jnp.float32),
        pltpu.VMEM_SHARED((_B // 16, 8, 128), jnp.float32),
        pltpu.SemaphoreType.DMA,
        pltpu.SemaphoreType.DMA,
    ],
)
def _center_loss_partials(label_hbm, feat_hbm, centers_t_hbm, out_hbm,
                          lab_v, mpk_v, bpk_v, cnt_s, slist_s, bufs,
                          fbuf_v, acc_v, feat_sh, sem, sem2):
    cid = lax.axis_index("c")
    sid = lax.axis_index("s")
    wid = sid * _NC + cid
    lanes = lax.iota(jnp.int32, _L)
    lane0 = lanes == 0

    @pl.when(sid == 0)
    def _():
        pltpu.sync_copy(feat_hbm, feat_sh)

    plsc.subcore_barrier()

    lo_strip = wid * _SPW
    lo = lo_strip * _SW
    hi = lo + _SPW * _SW
    zero = jnp.zeros((_L,), jnp.float32)

    def strip_src(s):
        s_eff = jnp.minimum(lo_strip + s, _LAST)
        off = pl.multiple_of(s_eff * _SW, _SW)
        return centers_t_hbm.at[:, pl.ds(off, _SW)]

    def wait_strip(buf):
        pltpu.make_async_copy(strip_src(jnp.int32(0)), buf, sem).wait()

    def segment(carry):
        h0, accs = carry

        def scan_chunk_cond(c):
            h, pos = c
            return (h < _NCHUNK) & (pos <= _CAP - _CHUNK)

        def scan_chunk(c):
            h, pos = c
            pltpu.sync_copy(label_hbm.at[pl.ds(h * _CHUNK, _CHUNK)], lab_v)

            def scan_body(k, pos):
                lv = lab_v[pl.ds(k * _L, _L)]
                m = (lv >= lo) & (lv < hi)
                pk = ((lv - lo) << 14) | (lanes + (h * _CHUNK + k * _L))
                plsc.store_compressed(mpk_v.at[pl.ds(pos, _L)], pk, mask=m)
                return pos + plsc.all_reduce_population_count(m)[0]

            pos = lax.fori_loop(0, _CHUNK // _L, scan_body, pos)
            return h + 1, pos

        h1, nm = lax.while_loop(scan_chunk_cond, scan_chunk,
                                (h0, jnp.int32(0)))

        def zero_body(s, carry):
            cnt_s[s] = 0
            return carry

        lax.fori_loop(0, _SPW + 1, zero_body, 0)

        def hist_body(m, carry):
            s = mpk_v[pl.ds(m, _L)][0] >> 21
            cnt_s[s] = cnt_s[s] + 1
            return carry

        lax.fori_loop(0, nm, hist_body, 0)

        def pfx_body(s, run):
            t = cnt_s[s]
            cnt_s[s] = run
            return run + t

        lax.fori_loop(0, _SPW + 1, pfx_body, jnp.int32(0))

        def ins_body(m, carry):
            pk = mpk_v[pl.ds(m, _L)][0]
            s = pk >> 21
            p = cnt_s[s]
            cnt_s[s] = p + 1
            plsc.store_scatter(bpk_v, [jnp.broadcast_to(p, (_L,))],
                               jnp.broadcast_to(pk, (_L,)), mask=lane0)
            return carry

        lax.fori_loop(0, nm, ins_body, 0)

        def build_body(s, carry):
            nu, prev = carry
            ce = cnt_s[s]

            @pl.when(ce > prev)
            def _():
                slist_s[nu] = s

            return jnp.where(ce > prev, nu + 1, nu), ce

        nu, _ = lax.fori_loop(0, _SPW, build_body,
                              (jnp.int32(0), jnp.int32(0)))

        @pl.when(nu == 0)
        def _():
            slist_s[0] = 0

        nu = jnp.maximum(nu, 1)

        def process(strip_v, ce, pe, accs):

            def mbody(m, accs):
                pk = bpk_v[pl.ds(m, _L)][0]
                r = pk & (_B - 1)
                cm = jnp.broadcast_to((pk >> 14) & (_SW - 1), (_L,))
                pltpu.sync_copy(feat_sh.at[r // 16, (r // 2) % 8], fbuf_v)
                foff = (r % 2) * _D
                new = []
                for q in range(_D // _L):
                    c = plsc.load_gather(strip_v, [lanes + q * _L, cm])
                    f = fbuf_v[pl.ds(foff + q * _L, _L)]
                    d = f - c
                    new.append(accs[q] + d * d)
                return tuple(new)

            accs = lax.fori_loop(pe, ce, mbody, accs)
            return ce, accs

        def sget(u):
            return slist_s[jnp.minimum(u, nu - 1)]

        for j in range(_NBUF):
            pltpu.async_copy(strip_src(sget(jnp.int32(j))), bufs[j], sem)

        def pipe_body(i, carry):
            pe, accs = carry
            u0 = i * _NBUF
            for j in range(_NBUF):
                wait_strip(bufs[j])
                pe, accs = process(bufs[j], cnt_s[sget(u0 + j)], pe, accs)
                pltpu.async_copy(strip_src(sget(u0 + j + _NBUF)), bufs[j],
                                 sem)
            return pe, accs

        ntr = (nu + _NBUF - 1) // _NBUF
        pe, accs = lax.fori_loop(0, ntr, pipe_body, (jnp.int32(0), accs))
        for j in range(_NBUF):
            wait_strip(bufs[j])

        return h1, accs

    def seg_cond(carry):
        h, _ = carry
        return h < _NCHUNK

    _, accs = lax.while_loop(seg_cond, segment,
                             (jnp.int32(0), (zero, zero, zero, zero)))

    zero16 = jnp.zeros((_L,), jnp.float32)
    for z in range(8):
        acc_v[pl.ds(z * _L, _L)] = zero16
    acc_v[pl.ds(0, _L)] = (accs[0] + accs[1]) + (accs[2] + accs[3])
    pltpu.sync_copy(acc_v, out_hbm.at[wid])


def kernel(label, feat, centers):
    feat2 = feat.reshape(_B // 16, 8, 128)
    partials = _center_loss_partials(label, feat2, centers.T)
    return jnp.sum(partials) * (0.5 / _B)
